# baseline (speedup 1.0000x reference)
import functools

import jax
import jax.numpy as jnp
from jax import lax
from jax.experimental import pallas as pl
from jax.experimental.pallas import tpu as pltpu
from jax.experimental.pallas import tpu_sc as plsc

_NC = 2
_NS = 16
_NW = _NC * _NS

_D = 2048
_NSEM = 4


def _gather_kernel(B):
    b_per_w = B // _NW
    mesh = plsc.VectorSubcoreMesh(core_axis_name="c", subcore_axis_name="s")

    @functools.partial(
        pl.kernel,
        mesh=mesh,
        out_type=jax.ShapeDtypeStruct((B, _D), jnp.float32),
        scratch_types=[
            pltpu.VMEM((b_per_w,), jnp.int32),
            pltpu.SemaphoreType.DMA((_NSEM,)),
        ],
    )
    def k(idx_hbm, table_hbm, out_hbm, idx_v, sems):
        wid = lax.axis_index("s") * _NC + lax.axis_index("c")
        base = wid * b_per_w
        pltpu.sync_copy(idx_hbm.at[pl.ds(base, b_per_w)], idx_v)

        def row_desc(row, i, q):
            return pltpu.make_async_copy(
                table_hbm.at[pl.ds(row, 1)],
                out_hbm.at[pl.ds(base + i, 1)],
                sems.at[q],
            )

        def issue(g, carry):
            v = idx_v[pl.ds(g * 16, 16)]
            for j in range(16):
                row = v[j]
                row_desc(row, g * 16 + j, j % _NSEM).start()
            return carry

        def drain(g, carry):
            for j in range(16):
                row_desc(0, g * 16 + j, j % _NSEM).wait()
            return carry

        n_grp = b_per_w // 16
        lax.fori_loop(0, n_grp, issue, 0)
        lax.fori_loop(0, n_grp, drain, 0)

    return k


@jax.jit
def kernel(x, table):
    B = x.shape[0] * x.shape[1]
    idx = x.reshape((B,)).astype(jnp.int32)
    out = _gather_kernel(B)(idx, table)
    return out.reshape(x.shape + (table.shape[1],))


# speedup vs baseline: 40.1142x; 40.1142x over previous
import functools

import jax
import jax.numpy as jnp
from jax import lax
from jax.experimental import pallas as pl
from jax.experimental.pallas import tpu as pltpu
from jax.experimental.pallas import tpu_sc as plsc

_NC = 2
_NS = 16
_NW = _NC * _NS

_D = 2048
_C = 8
_NBUF = 4


def _gather_kernel(B):
    b_per_w = B // _NW
    n_chunks = b_per_w // _C
    mesh = plsc.VectorSubcoreMesh(core_axis_name="c", subcore_axis_name="s")

    @functools.partial(
        pl.kernel,
        mesh=mesh,
        out_type=jax.ShapeDtypeStruct((B, _D), jnp.float32),
        scratch_types=[
            pltpu.VMEM((b_per_w,), jnp.int32),
            pltpu.VMEM((_NBUF, _C, _D), jnp.float32),
            pltpu.SemaphoreType.DMA,
            pltpu.SemaphoreType.DMA,
        ],
    )
    def k(idx_hbm, table_hbm, out_hbm, idx_v, rows_v, gsem, ssem):
        wid = lax.axis_index("s") * _NC + lax.axis_index("c")
        base = wid * b_per_w
        pltpu.sync_copy(idx_hbm.at[pl.ds(base, b_per_w)], idx_v)

        def gather_desc(ch, b):
            return pltpu.make_async_copy(
                table_hbm.at[idx_v.at[pl.ds(ch * _C, _C)]],
                rows_v.at[b],
                gsem,
            )

        def store_desc(ch, b):
            return pltpu.make_async_copy(
                rows_v.at[b],
                out_hbm.at[pl.ds(base + ch * _C, _C)],
                ssem,
            )

        for b in range(_NBUF):
            gather_desc(b, b).start()

        def body(it, carry):
            g = it * _NBUF
            for b in range(_NBUF):
                ch = g + b
                gather_desc(ch, b).wait()
                store_desc(ch, b).start()
            for b in range(_NBUF):
                ch = g + b
                nch = ch + _NBUF

                @pl.when(nch < n_chunks)
                def _():
                    store_desc(ch, b).wait()
                    gather_desc(nch, b).start()

            return carry

        lax.fori_loop(0, n_chunks // _NBUF, body, 0)
        for b in range(_NBUF):
            store_desc(0, b).wait()

    return k


@jax.jit
def kernel(x, table):
    B = x.shape[0] * x.shape[1]
    idx = x.reshape((B,)).astype(jnp.int32)
    out = _gather_kernel(B)(idx, table)
    return out.reshape(x.shape + (table.shape[1],))
